# baseline (speedup 1.0000x reference)
import functools

import jax
import jax.numpy as jnp
from jax import lax
from jax.experimental import pallas as pl
from jax.experimental.pallas import tpu as pltpu
from jax.experimental.pallas import tpu_sc as plsc

NUM_EXPERTS = 64
TOP_K = 6
AUX_COEF = 0.001
Z_COEF = 0.001


def _dense_body(hs_ref, gwt_ref, bias_ref,
                probs_ref, sump_ref, z_ref,
                sump_acc, z_acc):
    i = pl.program_id(0)
    n = pl.num_programs(0)

    @pl.when(i == 0)
    def _init():
        sump_acc[...] = jnp.zeros_like(sump_acc)
        z_acc[0, 0] = 0.0

    x = hs_ref[...]
    logits = jnp.dot(x, gwt_ref[...],
                     preferred_element_type=jnp.float32) + bias_ref[...]
    mx = jnp.max(logits, axis=-1, keepdims=True)
    ex = jnp.exp(logits - mx)
    sex = jnp.sum(ex, axis=-1, keepdims=True)
    probs = ex / sex
    probs_ref[...] = probs

    lse = mx + jnp.log(sex)
    z_acc[0, 0] += jnp.sum(lse * lse)
    sump_acc[...] += jnp.sum(probs, axis=0, keepdims=True)

    @pl.when(i == n - 1)
    def _fin():
        sump_ref[...] = sump_acc[...]
        z_ref[...] = jnp.reshape(z_acc[0, 0], (1, 1))


def _dense_call(hs2, gwt, bias2, t, h, blk):
    return pl.pallas_call(
        _dense_body,
        grid=(t // blk,),
        in_specs=[
            pl.BlockSpec((blk, h), lambda i: (i, 0)),
            pl.BlockSpec((h, NUM_EXPERTS), lambda i: (0, 0)),
            pl.BlockSpec((1, NUM_EXPERTS), lambda i: (0, 0)),
        ],
        out_specs=[
            pl.BlockSpec((blk, NUM_EXPERTS), lambda i: (i, 0)),
            pl.BlockSpec((1, NUM_EXPERTS), lambda i: (0, 0)),
            pl.BlockSpec((1, 1), lambda i: (0, 0)),
        ],
        out_shape=[
            jax.ShapeDtypeStruct((t, NUM_EXPERTS), jnp.float32),
            jax.ShapeDtypeStruct((1, NUM_EXPERTS), jnp.float32),
            jax.ShapeDtypeStruct((1, 1), jnp.float32),
        ],
        scratch_shapes=[
            pltpu.VMEM((1, NUM_EXPERTS), jnp.float32),
            pltpu.SMEM((1, 1), jnp.float32),
        ],
    )(hs2, gwt, bias2)


def _make_sc_topk(t, total_tokens):
    NW = 32
    tpw = t // NW
    ch = 256
    mesh = plsc.VectorSubcoreMesh(core_axis_name="c", subcore_axis_name="s")

    @functools.partial(
        pl.kernel, mesh=mesh,
        compiler_params=pltpu.CompilerParams(needs_layout_passes=False),
        out_type=[
            jax.ShapeDtypeStruct((t, 16), jnp.float32),
            jax.ShapeDtypeStruct((t, 16), jnp.int32),
            jax.ShapeDtypeStruct((32, NUM_EXPERTS), jnp.float32),
        ],
        scratch_types=[
            pltpu.VMEM((ch, NUM_EXPERTS), jnp.float32),
            pltpu.VMEM((ch, 16), jnp.float32),
            pltpu.VMEM((ch, 16), jnp.int32),
            pltpu.VMEM((NUM_EXPERTS,), jnp.float32),
        ],
    )
    def sc_topk(probs_hbm, rw_hbm, se_hbm, hist_hbm,
                pv, rwv, sev, hist):
        c = lax.axis_index("c")
        s = lax.axis_index("s")
        wid = s * 2 + c
        base = wid * tpw
        lane = lax.broadcasted_iota(jnp.int32, (16,), 0)
        ones = jnp.ones((16,), jnp.float32)

        for j in range(4):
            hist[pl.ds(16 * j, 16)] = jnp.zeros((16,), jnp.float32)

        def tok_body(tok, carry):
            keys = []
            for j in range(4):
                row = pv[tok, pl.ds(16 * j, 16)]
                rb = lax.bitcast_convert_type(row, jnp.int32)
                kb = (rb & ~63) | ((63 - 16 * j) - lane)
                keys.append(lax.bitcast_convert_type(kb, jnp.float32))
            wrow = jnp.zeros((16,), jnp.float32)
            serow = jnp.zeros((16,), jnp.int32)
            for k in range(TOP_K):
                mm = jnp.maximum(jnp.maximum(keys[0], keys[1]),
                                 jnp.maximum(keys[2], keys[3]))
                mval = jnp.max(mm)
                mvec = jnp.full((16,), mval)
                mb = lax.bitcast_convert_type(mvec, jnp.int32)
                hitk = lane == k
                wrow = jnp.where(
                    hitk,
                    lax.bitcast_convert_type(mb & ~63, jnp.float32), wrow)
                serow = jnp.where(hitk, 63 - (mb & 63), serow)
                for j in range(4):
                    keys[j] = jnp.where(keys[j] == mvec, -1.0, keys[j])
            rwv[tok, pl.ds(0, 16)] = wrow / jnp.full((16,), jnp.sum(wrow))
            sev[tok, pl.ds(0, 16)] = serow
            plsc.addupdate_scatter(hist, [serow], ones, mask=lane < TOP_K)
            return carry

        for ci in range(tpw // ch):
            cb = base + ci * ch
            pltpu.sync_copy(probs_hbm.at[pl.ds(cb, ch)], pv)
            lax.fori_loop(0, ch, tok_body, 0)
            pltpu.sync_copy(rwv, rw_hbm.at[pl.ds(cb, ch)])
            pltpu.sync_copy(sev, se_hbm.at[pl.ds(cb, ch)])

        pltpu.sync_copy(hist, hist_hbm.at[wid])

    return sc_topk


def _aux_body(hist_ref, sump_ref, z_ref, aux_ref, *, total_tokens):
    cnt = jnp.sum(hist_ref[...], axis=0, keepdims=True)
    frac = cnt / (jnp.sum(cnt) + 1e-9)
    avgp = jnp.sum(sump_ref[...], axis=0, keepdims=True) / total_tokens
    lbl = jnp.sum(frac * avgp) * NUM_EXPERTS
    aux_ref[...] = (AUX_COEF * jnp.reshape(lbl, (1, 1))
                    + Z_COEF * jnp.reshape(jnp.sum(z_ref[...]), (1, 1))
                    / total_tokens)


def _aux_call(hist32, sump, z, total_tokens):
    body = functools.partial(_aux_body, total_tokens=total_tokens)
    return pl.pallas_call(
        body,
        out_shape=jax.ShapeDtypeStruct((1, 1), jnp.float32),
    )(hist32, sump, z)


def kernel(hidden_states, pressure_bias, gate_weight):
    b, s, h = hidden_states.shape
    t = b * s
    hs2 = hidden_states.reshape(t, h)
    gwt = gate_weight.T
    bias2 = pressure_bias.reshape(1, NUM_EXPERTS)

    nslice = 4
    ts = t // nslice
    sc_call = _make_sc_topk(ts, float(t))
    probs_l, sump_l, z_l, rw_l, se_l, hist_l = [], [], [], [], [], []
    for si in range(nslice):
        hs_s = jax.lax.slice_in_dim(hs2, si * ts, (si + 1) * ts, axis=0)
        probs, sump, z = _dense_call(hs_s, gwt, bias2, ts, h, blk=4096)
        rw16, se16, hist32 = sc_call(probs)
        probs_l.append(probs)
        sump_l.append(sump)
        z_l.append(z)
        rw_l.append(rw16[:, :TOP_K])
        se_l.append(se16[:, :TOP_K])
        hist_l.append(hist32)

    aux = _aux_call(jnp.concatenate(hist_l, axis=0),
                    jnp.concatenate(sump_l, axis=0),
                    jnp.concatenate(z_l, axis=0), float(t))

    return (jnp.concatenate(rw_l, axis=0).reshape(b, s, TOP_K),
            jnp.concatenate(se_l, axis=0).reshape(b, s, TOP_K),
            jnp.concatenate(probs_l, axis=0).reshape(b, s, NUM_EXPERTS),
            aux.reshape(()))


# speedup vs baseline: 1.8816x; 1.8816x over previous
import functools

import jax
import jax.numpy as jnp
from jax import lax
from jax.experimental import pallas as pl
from jax.experimental.pallas import tpu as pltpu
from jax.experimental.pallas import tpu_sc as plsc

NUM_EXPERTS = 64
TOP_K = 6
AUX_COEF = 0.001
Z_COEF = 0.001


def _fused_body(hs_ref, gwt_ref, bias_ref,
                rw_ref, se_ref, probs_ref, sump_ref, z_ref,
                sump_acc, z_acc):
    i = pl.program_id(0)
    n = pl.num_programs(0)

    @pl.when(i == 0)
    def _init():
        sump_acc[...] = jnp.zeros_like(sump_acc)
        z_acc[0, 0] = 0.0

    x = hs_ref[...]
    logits = jnp.dot(x, gwt_ref[...],
                     preferred_element_type=jnp.float32) + bias_ref[...]
    mx = jnp.max(logits, axis=-1, keepdims=True)
    ex = jnp.exp(logits - mx)
    sex = jnp.sum(ex, axis=-1, keepdims=True)
    probs = ex / sex
    probs_ref[...] = probs

    lse = mx + jnp.log(sex)
    z_acc[0, 0] += jnp.sum(lse * lse)
    sump_acc[...] += jnp.sum(probs, axis=0, keepdims=True)

    lane = jax.lax.broadcasted_iota(jnp.int32, probs.shape, 1)
    pbits = jax.lax.bitcast_convert_type(probs, jnp.int32)
    key = jax.lax.bitcast_convert_type((pbits & ~63) | (63 - lane),
                                       jnp.float32)
    work = key
    cols = []
    for _ in range(TOP_K):
        m = jnp.max(work, axis=-1, keepdims=True)
        work = jnp.where(work == m, -1.0, work)
        cols.append(m)
    kcat = jnp.concatenate(cols, axis=1)
    kbits = jax.lax.bitcast_convert_type(kcat, jnp.int32)
    rw_ref[...] = (jax.lax.bitcast_convert_type(kbits & ~63, jnp.float32)
                   / jnp.sum(jax.lax.bitcast_convert_type(kbits & ~63,
                                                          jnp.float32),
                             axis=-1, keepdims=True))
    se_ref[...] = 63 - (kbits & 63)

    @pl.when(i == n - 1)
    def _fin():
        sump_ref[...] = sump_acc[...]
        z_ref[...] = jnp.reshape(z_acc[0, 0], (1, 1))


def _fused_call(hs2, gwt, bias2, t, h, blk):
    return pl.pallas_call(
        _fused_body,
        grid=(t // blk,),
        in_specs=[
            pl.BlockSpec((blk, h), lambda i: (i, 0)),
            pl.BlockSpec((h, NUM_EXPERTS), lambda i: (0, 0)),
            pl.BlockSpec((1, NUM_EXPERTS), lambda i: (0, 0)),
        ],
        out_specs=[
            pl.BlockSpec((blk, TOP_K), lambda i: (i, 0)),
            pl.BlockSpec((blk, TOP_K), lambda i: (i, 0)),
            pl.BlockSpec((blk, NUM_EXPERTS), lambda i: (i, 0)),
            pl.BlockSpec((1, NUM_EXPERTS), lambda i: (0, 0)),
            pl.BlockSpec((1, 1), lambda i: (0, 0)),
        ],
        out_shape=[
            jax.ShapeDtypeStruct((t, TOP_K), jnp.float32),
            jax.ShapeDtypeStruct((t, TOP_K), jnp.int32),
            jax.ShapeDtypeStruct((t, NUM_EXPERTS), jnp.float32),
            jax.ShapeDtypeStruct((1, NUM_EXPERTS), jnp.float32),
            jax.ShapeDtypeStruct((1, 1), jnp.float32),
        ],
        scratch_shapes=[
            pltpu.VMEM((1, NUM_EXPERTS), jnp.float32),
            pltpu.SMEM((1, 1), jnp.float32),
        ],
    )(hs2, gwt, bias2)


def _make_sc_count(n_entries):
    NW = 32
    epw = n_entries // NW
    mesh = plsc.VectorSubcoreMesh(core_axis_name="c", subcore_axis_name="s")

    @functools.partial(
        pl.kernel, mesh=mesh,
        compiler_params=pltpu.CompilerParams(needs_layout_passes=False),
        out_type=jax.ShapeDtypeStruct((NW, NUM_EXPERTS), jnp.float32),
        scratch_types=[
            pltpu.VMEM((epw,), jnp.int32),
            pltpu.VMEM((NUM_EXPERTS,), jnp.float32),
        ],
    )
    def sc_count(sef_hbm, hist_hbm, sv, hist):
        c = lax.axis_index("c")
        s = lax.axis_index("s")
        wid = s * 2 + c
        ones = jnp.ones((16,), jnp.float32)

        for j in range(4):
            hist[pl.ds(16 * j, 16)] = jnp.zeros((16,), jnp.float32)

        pltpu.sync_copy(sef_hbm.at[pl.ds(wid * epw, epw)], sv)

        def body(i, carry):
            v = sv[pl.ds(pl.multiple_of(i * 16, 16), 16)]
            plsc.addupdate_scatter(hist, [v], ones)
            return carry

        lax.fori_loop(0, epw // 16, body, 0)
        pltpu.sync_copy(hist, hist_hbm.at[wid])

    return sc_count


def _aux_body(hist_ref, sump_ref, z_ref, aux_ref, *, total_tokens):
    cnt = jnp.sum(hist_ref[...], axis=0, keepdims=True)
    frac = cnt / (jnp.sum(cnt) + 1e-9)
    avgp = sump_ref[...] / total_tokens
    lbl = jnp.sum(frac * avgp) * NUM_EXPERTS
    aux_ref[...] = (AUX_COEF * jnp.reshape(lbl, (1, 1))
                    + Z_COEF * z_ref[...] / total_tokens)


def _aux_call(hist32, sump, z, total_tokens):
    body = functools.partial(_aux_body, total_tokens=total_tokens)
    return pl.pallas_call(
        body,
        out_shape=jax.ShapeDtypeStruct((1, 1), jnp.float32),
    )(hist32, sump, z)


def kernel(hidden_states, pressure_bias, gate_weight):
    b, s, h = hidden_states.shape
    t = b * s
    hs2 = hidden_states.reshape(t, h)
    gwt = gate_weight.T
    bias2 = pressure_bias.reshape(1, NUM_EXPERTS)

    rw, se, probs, sump, z = _fused_call(hs2, gwt, bias2, t, h, blk=4096)

    hist32 = _make_sc_count(t * TOP_K)(se.reshape(t * TOP_K))
    aux = _aux_call(hist32, sump, z, float(t))

    return (rw.reshape(b, s, TOP_K),
            se.reshape(b, s, TOP_K),
            probs.reshape(b, s, NUM_EXPERTS),
            aux.reshape(()))
